# baseline (speedup 1.0000x reference)
import functools

import jax
import jax.numpy as jnp
from jax import lax
from jax.experimental import pallas as pl
from jax.experimental.pallas import tpu as pltpu
from jax.experimental.pallas import tpu_sc as plsc

_NUM_BINS = 15
_N = 8_000_000
_NC = 2
_NS = 16
_NW = _NC * _NS
_PER_TILE = _N // _NW
_CHUNK = 10_000
_NCHUNKS = _PER_TILE // _CHUNK
_VECS = _CHUNK // 16
_UNROLL = 8
_NBUF = 3
_ACC_W = 32


@functools.partial(
    pl.kernel,
    out_type=jax.ShapeDtypeStruct((2, _NW, 2 * 16 * _ACC_W), jnp.float32),
    mesh=plsc.VectorSubcoreMesh(core_axis_name="c", subcore_axis_name="s"),
    compiler_params=pltpu.CompilerParams(needs_layout_passes=False),
    scratch_types=[
        pltpu.VMEM((_CHUNK,), jnp.float32),
        pltpu.VMEM((_CHUNK,), jnp.float32),
        pltpu.VMEM((_CHUNK,), jnp.float32),
        pltpu.VMEM((_CHUNK,), jnp.int32),
        pltpu.VMEM((_CHUNK,), jnp.int32),
        pltpu.VMEM((_CHUNK,), jnp.int32),
        pltpu.VMEM((2 * 16 * _ACC_W,), jnp.float32),
        pltpu.VMEM((2 * 16 * _ACC_W,), jnp.float32),
        pltpu.SemaphoreType.DMA,
        pltpu.SemaphoreType.DMA,
        pltpu.SemaphoreType.DMA,
        pltpu.SemaphoreType.DMA,
        pltpu.SemaphoreType.DMA,
        pltpu.SemaphoreType.DMA,
    ],
)
def _ece_partials(probs_hbm, labels_hbm, out_hbm,
                  pb0, pb1, pb2, lb0, lb1, lb2, cnt_v, conf_v,
                  ps0, ps1, ps2, ls0, ls1, ls2):
    wid = lax.axis_index("s") * _NC + lax.axis_index("c")
    base = wid * _PER_TILE
    pbufs = (pb0, pb1, pb2)
    lbufs = (lb0, lb1, lb2)
    psem = (ps0, ps1, ps2)
    lsem = (ls0, ls1, ls2)

    zeros16 = jnp.zeros((16,), jnp.float32)
    for part in range(2 * _ACC_W):
        cnt_v[pl.ds(part * 16, 16)] = zeros16
        conf_v[pl.ds(part * 16, 16)] = zeros16

    lane = lax.broadcasted_iota(jnp.int32, (16,), 0)
    lane_h = (lane, lane + 16 * _ACC_W)
    ones = jnp.ones((16,), jnp.float32)

    def start_fetch(k):
        buf = k % _NBUF
        start = base + k * _CHUNK
        cp = pltpu.make_async_copy(
            probs_hbm.at[pl.ds(start, _CHUNK)], pbufs[buf], psem[buf])
        cl = pltpu.make_async_copy(
            labels_hbm.at[pl.ds(start, _CHUNK)], lbufs[buf], lsem[buf])
        cp.start()
        cl.start()
        return cp, cl

    pending = [start_fetch(0), start_fetch(1)]

    for k in range(_NCHUNKS):
        buf = k % _NBUF
        if k + 2 < _NCHUNKS:
            pending.append(start_fetch(k + 2))
        cp, cl = pending.pop(0)
        cp.wait()
        cl.wait()

        def do_vec(oo, half):
            p = pbufs[buf][pl.ds(oo, 16)]
            l = lbufs[buf][pl.ds(oo, 16)]
            ji = (p * jnp.float32(_NUM_BINS)).astype(jnp.int32)
            cidx = (ji + l * _NUM_BINS) * 16 + lane_h[half]
            plsc.addupdate_scatter(cnt_v, [cidx], ones)
            plsc.addupdate_scatter(conf_v, [cidx], p)

        @plsc.parallel_loop(0, _VECS // 2, step=1, unroll=_UNROLL)
        def body(i):
            o = i * 32
            do_vec(o, 0)
            do_vec(o + 16, 1)

        do_vec((_VECS - 1) * 16, 0)

    pltpu.sync_copy(cnt_v, out_hbm.at[0, wid])
    pltpu.sync_copy(conf_v, out_hbm.at[1, wid])


@jax.jit
def kernel(probs, labels):
    labels = labels.astype(jnp.int32)
    parts = _ece_partials(probs, labels)
    red = parts.reshape(2, _NW * 2, _ACC_W, 16).sum(axis=(1, 3))
    cnt2 = red[0]
    conf2 = red[1]
    nb = _NUM_BINS
    cnt_b = cnt2[:nb] + cnt2[nb:2 * nb]
    acc_b = cnt2[nb:2 * nb]
    conf_b = conf2[:nb] + conf2[nb:2 * nb]
    denom = jnp.maximum(cnt_b, 1.0)
    contrib = (cnt_b / _N) * jnp.abs(acc_b / denom - conf_b / denom)
    return jnp.sum(jnp.where(cnt_b > 0, contrib, 0.0))


# speedup vs baseline: 1.0796x; 1.0796x over previous
import functools

import jax
import jax.numpy as jnp
from jax import lax
from jax.experimental import pallas as pl
from jax.experimental.pallas import tpu as pltpu
from jax.experimental.pallas import tpu_sc as plsc

_NUM_BINS = 15
_N = 8_000_000
_NC = 2
_NS = 16
_NW = _NC * _NS
_PER_TILE = _N // _NW
_CHUNK = 20_000
_CHUNKS = [20_000] * 12 + [10_000]
_STARTS = [sum(_CHUNKS[:i]) for i in range(len(_CHUNKS))]
_NCHUNKS = len(_CHUNKS)
_UNROLL = 8
_ACC_W = 32


@functools.partial(
    pl.kernel,
    out_type=[
        jax.ShapeDtypeStruct((_NW, 16 * _ACC_W), jnp.float32),
        jax.ShapeDtypeStruct((_NW, 16 * _ACC_W), jnp.float32),
    ],
    mesh=plsc.VectorSubcoreMesh(core_axis_name="c", subcore_axis_name="s"),
    compiler_params=pltpu.CompilerParams(needs_layout_passes=False),
    scratch_types=[
        pltpu.VMEM((_CHUNK,), jnp.float32),
        pltpu.VMEM((_CHUNK,), jnp.float32),
        pltpu.VMEM((_CHUNK,), jnp.int32),
        pltpu.VMEM((_CHUNK,), jnp.int32),
        pltpu.VMEM((16 * _ACC_W,), jnp.float32),
        pltpu.VMEM((16 * _ACC_W,), jnp.float32),
        pltpu.SemaphoreType.DMA,
        pltpu.SemaphoreType.DMA,
        pltpu.SemaphoreType.DMA,
        pltpu.SemaphoreType.DMA,
    ],
)
def _ece_partials(probs_hbm, labels_hbm, cnt_out, conf_out,
                  pb0, pb1, lb0, lb1, cnt_v, conf_v,
                  ps0, ps1, ls0, ls1):
    wid = lax.axis_index("s") * _NC + lax.axis_index("c")
    base = wid * _PER_TILE
    pbufs = (pb0, pb1)
    lbufs = (lb0, lb1)
    psem = (ps0, ps1)
    lsem = (ls0, ls1)

    zeros16 = jnp.zeros((16,), jnp.float32)
    for part in range(_ACC_W):
        cnt_v[pl.ds(part * 16, 16)] = zeros16
        conf_v[pl.ds(part * 16, 16)] = zeros16

    lane = lax.broadcasted_iota(jnp.int32, (16,), 0)
    ones = jnp.ones((16,), jnp.float32)

    def start_fetch(k):
        buf = k % 2
        start = base + _STARTS[k]
        size = _CHUNKS[k]
        cp = pltpu.make_async_copy(
            probs_hbm.at[pl.ds(start, size)],
            pbufs[buf].at[pl.ds(0, size)], psem[buf])
        cl = pltpu.make_async_copy(
            labels_hbm.at[pl.ds(start, size)],
            lbufs[buf].at[pl.ds(0, size)], lsem[buf])
        cp.start()
        cl.start()
        return cp, cl

    pending = start_fetch(0)

    for k in range(_NCHUNKS):
        buf = k % 2
        cp, cl = pending
        cp.wait()
        cl.wait()
        if k + 1 < _NCHUNKS:
            pending = start_fetch(k + 1)

        @plsc.parallel_loop(0, _CHUNKS[k] // 16, step=1, unroll=_UNROLL)
        def body(i):
            o = i * 16
            p = pbufs[buf][pl.ds(o, 16)]
            l = lbufs[buf][pl.ds(o, 16)]
            ji = (p * jnp.float32(_NUM_BINS)).astype(jnp.int32)
            cidx = (ji + l * _NUM_BINS) * 16 + lane
            valid = p > 0.0
            plsc.addupdate_scatter(cnt_v, [cidx], ones, mask=valid)
            plsc.addupdate_scatter(conf_v, [cidx], p, mask=valid)

    pltpu.sync_copy(cnt_v, cnt_out.at[wid])
    pltpu.sync_copy(conf_v, conf_out.at[wid])


@jax.jit
def kernel(probs, labels):
    labels = labels.astype(jnp.int32)
    cnt_p, conf_p = _ece_partials(probs, labels)
    cnt2 = cnt_p.reshape(_NW, _ACC_W, 16).sum(axis=(0, 2))
    conf2 = conf_p.reshape(_NW, _ACC_W, 16).sum(axis=(0, 2))
    nb = _NUM_BINS
    cnt_b = cnt2[:nb] + cnt2[nb:2 * nb]
    acc_b = cnt2[nb:2 * nb]
    conf_b = conf2[:nb] + conf2[nb:2 * nb]
    denom = jnp.maximum(cnt_b, 1.0)
    contrib = (cnt_b / _N) * jnp.abs(acc_b / denom - conf_b / denom)
    return jnp.sum(jnp.where(cnt_b > 0, contrib, 0.0))


# speedup vs baseline: 1.0830x; 1.0031x over previous
import functools

import jax
import jax.numpy as jnp
from jax import lax
from jax.experimental import pallas as pl
from jax.experimental.pallas import tpu as pltpu
from jax.experimental.pallas import tpu_sc as plsc

_NUM_BINS = 15
_N = 8_000_000
_NC = 2
_NS = 16
_NW = _NC * _NS
_PER_TILE = _N // _NW
_CHUNK = 30_000
_CHUNKS = [30_000] * 8 + [10_000]
_STARTS = [sum(_CHUNKS[:i]) for i in range(len(_CHUNKS))]
_NCHUNKS = len(_CHUNKS)
_UNROLL = 8
_ACC_W = 32


@functools.partial(
    pl.kernel,
    out_type=[
        jax.ShapeDtypeStruct((_NW, 16 * _ACC_W), jnp.float32),
        jax.ShapeDtypeStruct((_NW, 16 * _ACC_W), jnp.float32),
    ],
    mesh=plsc.VectorSubcoreMesh(core_axis_name="c", subcore_axis_name="s"),
    compiler_params=pltpu.CompilerParams(needs_layout_passes=False),
    scratch_types=[
        pltpu.VMEM((_CHUNK,), jnp.float32),
        pltpu.VMEM((_CHUNK,), jnp.float32),
        pltpu.VMEM((_CHUNK,), jnp.int32),
        pltpu.VMEM((_CHUNK,), jnp.int32),
        pltpu.VMEM((16 * _ACC_W,), jnp.float32),
        pltpu.VMEM((16 * _ACC_W,), jnp.float32),
        pltpu.SemaphoreType.DMA,
        pltpu.SemaphoreType.DMA,
        pltpu.SemaphoreType.DMA,
        pltpu.SemaphoreType.DMA,
    ],
)
def _ece_partials(probs_hbm, labels_hbm, cnt_out, conf_out,
                  pb0, pb1, lb0, lb1, cnt_v, conf_v,
                  ps0, ps1, ls0, ls1):
    wid = lax.axis_index("s") * _NC + lax.axis_index("c")
    base = wid * _PER_TILE
    pbufs = (pb0, pb1)
    lbufs = (lb0, lb1)
    psem = (ps0, ps1)
    lsem = (ls0, ls1)

    zeros16 = jnp.zeros((16,), jnp.float32)
    for part in range(_ACC_W):
        cnt_v[pl.ds(part * 16, 16)] = zeros16
        conf_v[pl.ds(part * 16, 16)] = zeros16

    lane = lax.broadcasted_iota(jnp.int32, (16,), 0)
    ones = jnp.ones((16,), jnp.float32)

    def start_fetch(k):
        buf = k % 2
        start = base + _STARTS[k]
        size = _CHUNKS[k]
        cp = pltpu.make_async_copy(
            probs_hbm.at[pl.ds(start, size)],
            pbufs[buf].at[pl.ds(0, size)], psem[buf])
        cl = pltpu.make_async_copy(
            labels_hbm.at[pl.ds(start, size)],
            lbufs[buf].at[pl.ds(0, size)], lsem[buf])
        cp.start()
        cl.start()
        return cp, cl

    pending = start_fetch(0)

    for k in range(_NCHUNKS):
        buf = k % 2
        cp, cl = pending
        cp.wait()
        cl.wait()
        if k + 1 < _NCHUNKS:
            pending = start_fetch(k + 1)

        @plsc.parallel_loop(0, _CHUNKS[k] // 16, step=1, unroll=_UNROLL)
        def body(i):
            o = i * 16
            p = pbufs[buf][pl.ds(o, 16)]
            l = lbufs[buf][pl.ds(o, 16)]
            ji = (p * jnp.float32(_NUM_BINS)).astype(jnp.int32)
            cidx = (ji + l * _NUM_BINS) * 16 + lane
            valid = p > 0.0
            plsc.addupdate_scatter(cnt_v, [cidx], ones, mask=valid)
            plsc.addupdate_scatter(conf_v, [cidx], p, mask=valid)

    pltpu.sync_copy(cnt_v, cnt_out.at[wid])
    pltpu.sync_copy(conf_v, conf_out.at[wid])


@jax.jit
def kernel(probs, labels):
    labels = labels.astype(jnp.int32)
    cnt_p, conf_p = _ece_partials(probs, labels)
    cnt2 = cnt_p.reshape(_NW, _ACC_W, 16).sum(axis=(0, 2))
    conf2 = conf_p.reshape(_NW, _ACC_W, 16).sum(axis=(0, 2))
    nb = _NUM_BINS
    cnt_b = cnt2[:nb] + cnt2[nb:2 * nb]
    acc_b = cnt2[nb:2 * nb]
    conf_b = conf2[:nb] + conf2[nb:2 * nb]
    denom = jnp.maximum(cnt_b, 1.0)
    contrib = (cnt_b / _N) * jnp.abs(acc_b / denom - conf_b / denom)
    return jnp.sum(jnp.where(cnt_b > 0, contrib, 0.0))


# speedup vs baseline: 1.1162x; 1.0306x over previous
import functools

import jax
import jax.numpy as jnp
from jax import lax
from jax.experimental import pallas as pl
from jax.experimental.pallas import tpu as pltpu
from jax.experimental.pallas import tpu_sc as plsc

_NUM_BINS = 15
_N = 8_000_000
_NC = 2
_NS = 16
_NW = _NC * _NS
_PER_TILE = _N // _NW
_CHUNK = 30_000
_CHUNKS = [30_000] * 8 + [10_000]
_STARTS = [sum(_CHUNKS[:i]) for i in range(len(_CHUNKS))]
_NCHUNKS = len(_CHUNKS)
_UNROLL = 8
_ACC_W = 32


@functools.partial(
    pl.kernel,
    out_type=[
        jax.ShapeDtypeStruct((_NW, 16 * _ACC_W), jnp.float32),
        jax.ShapeDtypeStruct((_NW, 16 * _ACC_W), jnp.float32),
    ],
    mesh=plsc.VectorSubcoreMesh(core_axis_name="c", subcore_axis_name="s"),
    compiler_params=pltpu.CompilerParams(needs_layout_passes=False),
    scratch_types=[
        pltpu.VMEM((_CHUNK,), jnp.float32),
        pltpu.VMEM((_CHUNK,), jnp.float32),
        pltpu.VMEM((_CHUNK,), jnp.int32),
        pltpu.VMEM((_CHUNK,), jnp.int32),
        pltpu.VMEM((16 * _ACC_W,), jnp.float32),
        pltpu.VMEM((16 * _ACC_W,), jnp.float32),
        pltpu.SemaphoreType.DMA,
        pltpu.SemaphoreType.DMA,
        pltpu.SemaphoreType.DMA,
        pltpu.SemaphoreType.DMA,
    ],
)
def _ece_partials(probs_hbm, labels_hbm, cnt_out, conf_out,
                  pb0, pb1, lb0, lb1, cnt_v, conf_v,
                  ps0, ps1, ls0, ls1):
    wid = lax.axis_index("s") * _NC + lax.axis_index("c")
    base = wid * _PER_TILE
    pbufs = (pb0, pb1)
    lbufs = (lb0, lb1)
    psem = (ps0, ps1)
    lsem = (ls0, ls1)

    zeros16 = jnp.zeros((16,), jnp.float32)
    for part in range(_ACC_W):
        cnt_v[pl.ds(part * 16, 16)] = zeros16
        conf_v[pl.ds(part * 16, 16)] = zeros16

    lane = lax.broadcasted_iota(jnp.int32, (16,), 0)
    ones = jnp.ones((16,), jnp.float32)

    def start_fetch(k):
        buf = k % 2
        start = pl.multiple_of(base + _STARTS[k], 16)
        size = _CHUNKS[k]
        cp = pltpu.make_async_copy(
            probs_hbm.at[pl.ds(start, size)],
            pbufs[buf].at[pl.ds(0, size)], psem[buf])
        cl = pltpu.make_async_copy(
            labels_hbm.at[pl.ds(start, size)],
            lbufs[buf].at[pl.ds(0, size)], lsem[buf])
        cp.start()
        cl.start()
        return cp, cl

    pending = start_fetch(0)

    for k in range(_NCHUNKS):
        buf = k % 2
        cp, cl = pending
        cp.wait()
        cl.wait()
        if k + 1 < _NCHUNKS:
            pending = start_fetch(k + 1)

        @plsc.parallel_loop(0, _CHUNKS[k] // 16, step=1, unroll=_UNROLL)
        def body(i):
            o = i * 16
            p = pbufs[buf][pl.ds(o, 16)]
            l = lbufs[buf][pl.ds(o, 16)]
            ji = (p * jnp.float32(_NUM_BINS)).astype(jnp.int32)
            cidx = (ji << 4) | (l << 8) | lane
            valid = p > 0.0
            plsc.addupdate_scatter(cnt_v, [cidx], ones, mask=valid)
            plsc.addupdate_scatter(conf_v, [cidx], p, mask=valid)

    pltpu.sync_copy(cnt_v, cnt_out.at[wid])
    pltpu.sync_copy(conf_v, conf_out.at[wid])


@jax.jit
def kernel(probs, labels):
    labels = labels.astype(jnp.int32)
    cnt_p, conf_p = _ece_partials(probs, labels)
    cnt2 = cnt_p.reshape(_NW, _ACC_W, 16).sum(axis=(0, 2))
    conf2 = conf_p.reshape(_NW, _ACC_W, 16).sum(axis=(0, 2))
    nb = _NUM_BINS
    cnt_b = cnt2[:nb] + cnt2[16:16 + nb]
    acc_b = cnt2[16:16 + nb]
    conf_b = conf2[:nb] + conf2[16:16 + nb]
    denom = jnp.maximum(cnt_b, 1.0)
    contrib = (cnt_b / _N) * jnp.abs(acc_b / denom - conf_b / denom)
    return jnp.sum(jnp.where(cnt_b > 0, contrib, 0.0))
